# baseline (speedup 1.0000x reference)
import functools

import jax
import jax.numpy as jnp
from jax import lax
from jax.experimental import pallas as pl
from jax.experimental.pallas import tpu as pltpu
from jax.experimental.pallas import tpu_sc as plsc

B = 16384
EMB = 64
NC = 2
NS = 16
NW = NC * NS
BPW = B // NW
CHUNK = 128
NCHUNK = BPW // CHUNK

_mesh = plsc.VectorSubcoreMesh(core_axis_name="c", subcore_axis_name="s")


@functools.partial(
    pl.kernel,
    out_type=jax.ShapeDtypeStruct((B,), jnp.float32),
    mesh=_mesh,
    compiler_params=pltpu.CompilerParams(needs_layout_passes=False,
                                         use_tc_tiling_on_sc=False),
    scratch_types=[
        pltpu.VMEM((NCHUNK, CHUNK), jnp.int32),
        pltpu.VMEM((NCHUNK, CHUNK), jnp.int32),
        pltpu.VMEM((NCHUNK, CHUNK, EMB), jnp.float32),
        pltpu.VMEM((NCHUNK, CHUNK, EMB), jnp.float32),
        pltpu.VMEM((BPW * 16,), jnp.float32),
        pltpu.VMEM((BPW,), jnp.float32),
        pltpu.SemaphoreType.DMA,
    ],
)
def _dot_kernel(users_hbm, items_hbm, uemb_hbm, iemb_hbm, out_hbm,
                uidx, iidx, urows, irows, part, outb, sem):
    wid = lax.axis_index("s") * NC + lax.axis_index("c")
    base = wid * BPW

    for j in range(NCHUNK):
        pltpu.sync_copy(users_hbm.at[pl.ds(base + j * CHUNK, CHUNK)],
                        uidx.at[j])
        pltpu.sync_copy(items_hbm.at[pl.ds(base + j * CHUNK, CHUNK)],
                        iidx.at[j])

    handles = []
    for j in range(NCHUNK):
        handles.append(pltpu.async_copy(uemb_hbm.at[uidx.at[j]],
                                        urows.at[j], sem))
        handles.append(pltpu.async_copy(iemb_hbm.at[iidx.at[j]],
                                        irows.at[j], sem))
    for h in handles:
        h.wait()

    for j in range(NCHUNK):
        def row_body(r, _, j=j):
            acc = (urows[j, r, pl.ds(0, 16)] * irows[j, r, pl.ds(0, 16)])
            for k in range(1, EMB // 16):
                acc = acc + (urows[j, r, pl.ds(k * 16, 16)]
                             * irows[j, r, pl.ds(k * 16, 16)])
            part[pl.ds((j * CHUNK + r) * 16, 16)] = acc
            return 0
        lax.fori_loop(0, CHUNK, row_body, 0)

    iota16 = lax.iota(jnp.int32, 16)

    def grp_body(g, _):
        row0 = g * 16
        vec0 = row0 * 16 + iota16 * 16
        acc = plsc.load_gather(part, [vec0])
        for l in range(1, 16):
            acc = acc + plsc.load_gather(part, [vec0 + l])
        outb[pl.ds(row0, 16)] = acc
        return 0
    lax.fori_loop(0, BPW // 16, grp_body, 0)

    pltpu.sync_copy(outb, out_hbm.at[pl.ds(base, BPW)])


@functools.partial(
    pl.kernel,
    out_type=jax.ShapeDtypeStruct((B,), jnp.float32),
    mesh=_mesh,
    compiler_params=pltpu.CompilerParams(needs_layout_passes=False),
    scratch_types=[
        pltpu.SMEM((BPW,), jnp.int32),
        pltpu.SMEM((BPW,), jnp.int32),
        pltpu.VMEM((BPW,), jnp.int32),
        pltpu.VMEM((BPW,), jnp.int32),
        pltpu.VMEM((32 * 8, 1), jnp.float32),
        pltpu.VMEM((32 * 8, 1), jnp.float32),
        pltpu.VMEM((BPW,), jnp.float32),
        pltpu.VMEM((BPW,), jnp.float32),
        pltpu.SemaphoreType.DMA,
    ],
)
def _bias_kernel(users_hbm, items_hbm, ubias_hbm, ibias_hbm, dot_hbm,
                 out_hbm, uidx, iidx, uidx_v, iidx_v, ubstage, ibstage,
                 dotv, outb, sem):
    wid = lax.axis_index("s") * NC + lax.axis_index("c")
    base = wid * BPW

    pltpu.sync_copy(users_hbm.at[pl.ds(base, BPW)], uidx_v)
    pltpu.sync_copy(items_hbm.at[pl.ds(base, BPW)], iidx_v)
    pltpu.sync_copy(dot_hbm.at[pl.ds(base, BPW)], dotv)

    iota16 = lax.iota(jnp.int32, 16)
    zeros16 = jnp.zeros((16,), jnp.int32)

    def extract_body(v, _):
        uv = uidx_v[pl.ds(v * 16, 16)]
        iv = iidx_v[pl.ds(v * 16, 16)]
        for l in range(16):
            m = iota16 == l
            uidx[v * 16 + l] = jnp.sum(jnp.where(m, uv, 0))
            iidx[v * 16 + l] = jnp.sum(jnp.where(m, iv, 0))
        return 0
    lax.fori_loop(0, BPW // 16, extract_body, 0)

    for c in range(BPW // 32):
        def fire(r, _, c=c):
            g = c * 32 + r
            u = uidx[g]
            it = iidx[g]
            pltpu.async_copy(ubias_hbm.at[pl.ds((u >> 3) * 8, 8)],
                             ubstage.at[pl.ds(r * 8, 8)], sem)
            pltpu.async_copy(ibias_hbm.at[pl.ds((it >> 3) * 8, 8)],
                             ibstage.at[pl.ds(r * 8, 8)], sem)
            return 0
        lax.fori_loop(0, 32, fire, 0)

        def drain(r, _, c=c):
            g = c * 32 + r
            u = uidx[g]
            it = iidx[g]
            pltpu.make_async_copy(ubias_hbm.at[pl.ds((u >> 3) * 8, 8)],
                                  ubstage.at[pl.ds(r * 8, 8)], sem).wait()
            pltpu.make_async_copy(ibias_hbm.at[pl.ds((it >> 3) * 8, 8)],
                                  ibstage.at[pl.ds(r * 8, 8)], sem).wait()
            return 0
        lax.fori_loop(0, 32, drain, 0)

        def grp_body(g, _, c=c):
            row0 = g * 16
            grow0 = c * 32 + row0
            usub = uidx_v[pl.ds(grow0, 16)] & 7
            isub = iidx_v[pl.ds(grow0, 16)] & 7
            srows = (row0 + iota16) * 8
            ubv = plsc.load_gather(ubstage, [srows + usub, zeros16])
            ibv = plsc.load_gather(ibstage, [srows + isub, zeros16])
            outb[pl.ds(grow0, 16)] = dotv[pl.ds(grow0, 16)] + ubv + ibv
            return 0
        lax.fori_loop(0, 32 // 16, grp_body, 0)

    pltpu.sync_copy(outb, out_hbm.at[pl.ds(base, BPW)])


def kernel(users, items, user_emb, item_emb, user_bias, item_bias):
    users = users.astype(jnp.int32)
    items = items.astype(jnp.int32)
    dots = _dot_kernel(users, items, user_emb, item_emb)
    return _bias_kernel(users, items, user_bias, item_bias, dots)


# speedup vs baseline: 1.3224x; 1.3224x over previous
import functools

import jax
import jax.numpy as jnp
from jax import lax
from jax.experimental import pallas as pl
from jax.experimental.pallas import tpu as pltpu
from jax.experimental.pallas import tpu_sc as plsc

B = 16384
EMB = 64
NC = 2
NS = 16
NW = NC * NS
BPW = B // NW
CHUNK = 128
NCHUNK = BPW // CHUNK

_mesh = plsc.VectorSubcoreMesh(core_axis_name="c", subcore_axis_name="s")
_params = pltpu.CompilerParams(needs_layout_passes=False,
                               use_tc_tiling_on_sc=False)


@functools.partial(
    pl.kernel,
    out_type=jax.ShapeDtypeStruct((B, EMB), jnp.float32),
    mesh=_mesh,
    compiler_params=_params,
    scratch_types=[
        pltpu.VMEM((NCHUNK, CHUNK), jnp.int32),
        pltpu.VMEM((NCHUNK, CHUNK, EMB), jnp.float32),
        pltpu.SemaphoreType.DMA,
    ],
)
def _row_gather(idx_hbm, emb_hbm, out_hbm, idx, rows, sem):
    wid = lax.axis_index("s") * NC + lax.axis_index("c")
    base = wid * BPW

    for j in range(NCHUNK):
        pltpu.sync_copy(idx_hbm.at[pl.ds(base + j * CHUNK, CHUNK)],
                        idx.at[j])
    handles = [pltpu.async_copy(emb_hbm.at[idx.at[j]], rows.at[j], sem)
               for j in range(NCHUNK)]
    for h in handles:
        h.wait()
    for j in range(NCHUNK):
        pltpu.sync_copy(rows.at[j],
                        out_hbm.at[pl.ds(base + j * CHUNK, CHUNK)])


@functools.partial(
    pl.kernel,
    out_type=jax.ShapeDtypeStruct((B,), jnp.float32),
    mesh=_mesh,
    compiler_params=_params,
    scratch_types=[
        pltpu.VMEM((NCHUNK, CHUNK), jnp.int32),
        pltpu.VMEM((NCHUNK, CHUNK), jnp.int32),
        pltpu.VMEM((NCHUNK, CHUNK, EMB), jnp.float32),
        pltpu.VMEM((NCHUNK, CHUNK, EMB), jnp.float32),
        pltpu.VMEM((NCHUNK, CHUNK), jnp.float32),
        pltpu.VMEM((NCHUNK, CHUNK), jnp.float32),
        pltpu.VMEM((BPW * 16,), jnp.float32),
        pltpu.VMEM((BPW,), jnp.float32),
        pltpu.SemaphoreType.DMA,
    ],
)
def _dot_bias(users_hbm, items_hbm, rowsu_hbm, rowsi_hbm, ubias_hbm,
              ibias_hbm, out_hbm, uidx, iidx, urows, irows, ub, ib,
              part, outb, sem):
    wid = lax.axis_index("s") * NC + lax.axis_index("c")
    base = wid * BPW

    for j in range(NCHUNK):
        pltpu.sync_copy(users_hbm.at[pl.ds(base + j * CHUNK, CHUNK)],
                        uidx.at[j])
        pltpu.sync_copy(items_hbm.at[pl.ds(base + j * CHUNK, CHUNK)],
                        iidx.at[j])

    handles = []
    for j in range(NCHUNK):
        handles.append(pltpu.async_copy(
            rowsu_hbm.at[pl.ds(base + j * CHUNK, CHUNK)], urows.at[j],
            sem))
        handles.append(pltpu.async_copy(
            rowsi_hbm.at[pl.ds(base + j * CHUNK, CHUNK)], irows.at[j],
            sem))
        handles.append(pltpu.async_copy(ubias_hbm.at[uidx.at[j]],
                                        ub.at[j], sem))
        handles.append(pltpu.async_copy(ibias_hbm.at[iidx.at[j]],
                                        ib.at[j], sem))
    for h in handles:
        h.wait()

    for j in range(NCHUNK):
        def row_body(r, _, j=j):
            acc = (urows[j, r, pl.ds(0, 16)] * irows[j, r, pl.ds(0, 16)])
            for k in range(1, EMB // 16):
                acc = acc + (urows[j, r, pl.ds(k * 16, 16)]
                             * irows[j, r, pl.ds(k * 16, 16)])
            part[pl.ds((j * CHUNK + r) * 16, 16)] = acc
            return 0
        lax.fori_loop(0, CHUNK, row_body, 0)

    iota16 = lax.iota(jnp.int32, 16)

    for j in range(NCHUNK):
        def grp_body(g, _, j=j):
            row0 = j * CHUNK + g * 16
            vec0 = row0 * 16 + iota16 * 16
            acc = plsc.load_gather(part, [vec0])
            for l in range(1, 16):
                acc = acc + plsc.load_gather(part, [vec0 + l])
            outb[pl.ds(row0, 16)] = (acc + ub[j, pl.ds(g * 16, 16)]
                                     + ib[j, pl.ds(g * 16, 16)])
            return 0
        lax.fori_loop(0, CHUNK // 16, grp_body, 0)

    pltpu.sync_copy(outb, out_hbm.at[pl.ds(base, BPW)])


def kernel(users, items, user_emb, item_emb, user_bias, item_bias):
    users = users.astype(jnp.int32)
    items = items.astype(jnp.int32)
    rows_u = _row_gather(users, user_emb)
    rows_i = _row_gather(items, item_emb)
    return _dot_bias(users, items, rows_u, rows_i,
                     user_bias.reshape(-1), item_bias.reshape(-1))


# speedup vs baseline: 1.4246x; 1.0773x over previous
import functools

import jax
import jax.numpy as jnp
from jax import lax
from jax.experimental import pallas as pl
from jax.experimental.pallas import tpu as pltpu
from jax.experimental.pallas import tpu_sc as plsc

B = 16384
EMB = 64
NC = 2
NS = 16
NW = NC * NS
BPW = B // NW
CHUNK = 128
NCHUNK = BPW // CHUNK

_mesh = plsc.VectorSubcoreMesh(core_axis_name="c", subcore_axis_name="s")


@functools.partial(
    pl.kernel,
    out_type=jax.ShapeDtypeStruct((B,), jnp.float32),
    mesh=_mesh,
    compiler_params=pltpu.CompilerParams(needs_layout_passes=False,
                                         use_tc_tiling_on_sc=False),
    scratch_types=[
        pltpu.VMEM((NCHUNK, CHUNK), jnp.int32),
        pltpu.VMEM((NCHUNK, CHUNK), jnp.int32),
        pltpu.VMEM((NCHUNK, CHUNK, EMB), jnp.float32),
        pltpu.VMEM((NCHUNK, CHUNK, EMB), jnp.float32),
        pltpu.VMEM((NCHUNK, CHUNK), jnp.float32),
        pltpu.VMEM((NCHUNK, CHUNK), jnp.float32),
        pltpu.VMEM((BPW * 16,), jnp.float32),
        pltpu.VMEM((BPW,), jnp.float32),
        pltpu.SemaphoreType.DMA,
    ],
)
def _sc_kernel(users_hbm, items_hbm, uemb_hbm, iemb_hbm, ubias_hbm,
               ibias_hbm, out_hbm, uidx, iidx, urows, irows, ub, ib,
               part, outb, sem):
    wid = lax.axis_index("s") * NC + lax.axis_index("c")
    base = wid * BPW

    for j in range(NCHUNK):
        pltpu.sync_copy(users_hbm.at[pl.ds(base + j * CHUNK, CHUNK)],
                        uidx.at[j])
        pltpu.sync_copy(items_hbm.at[pl.ds(base + j * CHUNK, CHUNK)],
                        iidx.at[j])

    handles = []
    for j in range(NCHUNK):
        handles.append(pltpu.async_copy(uemb_hbm.at[uidx.at[j]],
                                        urows.at[j], sem))
        handles.append(pltpu.async_copy(iemb_hbm.at[iidx.at[j]],
                                        irows.at[j], sem))
        handles.append(pltpu.async_copy(ubias_hbm.at[uidx.at[j]],
                                        ub.at[j], sem))
        handles.append(pltpu.async_copy(ibias_hbm.at[iidx.at[j]],
                                        ib.at[j], sem))
    for h in handles:
        h.wait()

    for j in range(NCHUNK):
        def row_body(r, _, j=j):
            acc = (urows[j, r, pl.ds(0, 16)] * irows[j, r, pl.ds(0, 16)])
            for k in range(1, EMB // 16):
                acc = acc + (urows[j, r, pl.ds(k * 16, 16)]
                             * irows[j, r, pl.ds(k * 16, 16)])
            part[pl.ds((j * CHUNK + r) * 16, 16)] = acc
            return 0
        lax.fori_loop(0, CHUNK, row_body, 0)

    iota16 = lax.iota(jnp.int32, 16)
    for j in range(NCHUNK):
        def grp_body(g, _, j=j):
            row0 = j * CHUNK + g * 16
            vec0 = row0 * 16 + iota16 * 16
            acc = plsc.load_gather(part, [vec0])
            for l in range(1, 16):
                acc = acc + plsc.load_gather(part, [vec0 + l])
            res = acc + ub[j, pl.ds(g * 16, 16)] + ib[j, pl.ds(g * 16, 16)]
            outb[pl.ds(row0, 16)] = res
            return 0
        lax.fori_loop(0, CHUNK // 16, grp_body, 0)

    pltpu.sync_copy(outb, out_hbm.at[pl.ds(base, BPW)])


def kernel(users, items, user_emb, item_emb, user_bias, item_bias):
    return _sc_kernel(users.astype(jnp.int32), items.astype(jnp.int32),
                      user_emb, item_emb, user_bias.reshape(-1),
                      item_bias.reshape(-1))
